# baseline (speedup 1.0000x reference)
import functools

import jax
import jax.numpy as jnp
from jax import lax
from jax.experimental import pallas as pl
from jax.experimental.pallas import tpu as pltpu
from jax.experimental.pallas import tpu_sc as plsc

_V = 1000000
_D = 64
_MAX_NORM = 1.0
_B = 16384

_NW = 32
_NCOLS = (_V + 127) // 128
_CPW = (_NCOLS + _NW - 1) // _NW
_IDS_CHUNK = 2048
_OUTW = _B
_RING = 64
_BW = 256


def _iota16():
    return jnp.arange(16, dtype=jnp.int32)


def _make_sc_kernel():
    mesh = plsc.VectorSubcoreMesh(core_axis_name="c", subcore_axis_name="s")

    @functools.partial(
        pl.kernel,
        mesh=mesh,
        compiler_params=pltpu.CompilerParams(
            use_tc_tiling_on_sc=True, needs_layout_passes=False),
        out_type=jax.ShapeDtypeStruct((_D * _OUTW,), jnp.float32),
        scratch_types=[
            pltpu.VMEM((2, _D, _BW), jnp.float32),
            pltpu.VMEM((_CPW * 128,), jnp.int32),
            pltpu.VMEM((_IDS_CHUNK,), jnp.int32),
            pltpu.VMEM((_RING, _D), jnp.float32),
            pltpu.VMEM((_RING, _D), jnp.int32),
            pltpu.VMEM((_B,), jnp.int32),
            pltpu.VMEM((_B,), jnp.int32),
            pltpu.VMEM((_D,), jnp.float32),
            pltpu.SemaphoreType.DMA,
            pltpu.SemaphoreType.DMA,
            pltpu.SemaphoreType.DMA,
        ],
    )
    def sc_kernel(ids_hbm, tabT_hbm, raw_hbm, blk_v, r_v, ids_v,
                  srcr_v, idxr_v, dupd_v, dupi_v, dummy_v,
                  bsem0, bsem1, ssem):
        wid = lax.axis_index("s") * 2 + lax.axis_index("c")
        lo_col = wid * _CPW
        hi_col = jnp.minimum(lo_col + _CPW, _NCOLS)
        ncols = hi_col - lo_col
        lo_id = lo_col * 128
        hi_id = jnp.minimum(hi_col * 128, _V)
        iota = _iota16()
        zero16 = jnp.zeros((16,), jnp.int32)

        def z_body(i, _):
            for u in range(4):
                r_v[pl.ds((i * 4 + u) * 16, 16)] = zero16
            return 0
        lax.fori_loop(0, _CPW * 128 // 64, z_body, 0)

        def scan1_chunk(g, _):
            pltpu.sync_copy(ids_hbm.at[pl.ds(g * _IDS_CHUNK, _IDS_CHUNK)],
                            ids_v)
            def scan1_vec(k, _):
                for u in range(4):
                    o = (k * 4 + u) * 16
                    v = ids_v[pl.ds(o, 16)]
                    jv = g * _IDS_CHUNK + o + iota
                    m = (v >= lo_id) & (v < hi_id)
                    plsc.store_scatter(r_v, [v - lo_id], jv + 1, mask=m)
                return 0
            lax.fori_loop(0, _IDS_CHUNK // 64, scan1_vec, 0)
            return 0
        lax.fori_loop(0, _B // _IDS_CHUNK, scan1_chunk, 0)

        def scan2_chunk(g, dcnt):
            pltpu.sync_copy(ids_hbm.at[pl.ds(g * _IDS_CHUNK, _IDS_CHUNK)],
                            ids_v)
            def scan2_vec(k, dcnt):
                for u in range(4):
                    o = (k * 4 + u) * 16
                    v = ids_v[pl.ds(o, 16)]
                    jv = g * _IDS_CHUNK + o + iota
                    m = (v >= lo_id) & (v < hi_id)
                    rb = plsc.load_gather(r_v,
                                          [jnp.where(m, v - lo_id, 0)])
                    coll = m & (rb != jv + 1)
                    pos = plsc.cumsum(coll.astype(jnp.int32)) - 1 + dcnt
                    plsc.store_scatter(dupd_v, [pos], jv, mask=coll)
                    plsc.store_scatter(dupi_v, [pos], v, mask=coll)
                    n = plsc.all_reduce_population_count(coll)
                    dcnt = dcnt + n[0]
                return dcnt
            return lax.fori_loop(0, _IDS_CHUNK // 64, scan2_vec, dcnt)
        dupcnt = lax.fori_loop(0, _B // _IDS_CHUNK, scan2_chunk,
                               jnp.int32(0))

        _BSEMS = (bsem0, bsem1)
        span = ncols * 128
        nblk = (span + _BW - 1) // _BW

        def blk_off(c):
            return pl.multiple_of(jnp.minimum(c * _BW, span - _BW), 128)

        def start_blk(c, par):
            col = pl.multiple_of(lo_id + blk_off(c), 128)
            return pltpu.async_copy(
                tabT_hbm.at[:, pl.ds(col, _BW)], blk_v.at[par],
                _BSEMS[par])

        def wait_blk(par):
            col0 = pl.multiple_of(0, 128)
            pltpu.make_async_copy(
                tabT_hbm.at[:, pl.ds(col0, _BW)], blk_v.at[par],
                _BSEMS[par]).wait()

        @pl.when(nblk > 0)
        def _():
            start_blk(0, 0)

        @pl.when(nblk > 1)
        def _():
            start_blk(1, 1)

        def extract_one(l, jval, par, mc):
            slot = lax.rem(mc, _RING)

            @pl.when(mc >= _RING)
            def _():
                pltpu.make_async_copy(
                    raw_hbm.at[pl.ds(0, _D)], dummy_v, ssem).wait()

            lvec = jnp.full((16,), l, jnp.int32)
            for q in range(4):
                dv = iota + q * 16
                srcr_v[slot, pl.ds(q * 16, 16)] = plsc.load_gather(
                    blk_v.at[par], [dv, lvec])
                idxr_v[slot, pl.ds(q * 16, 16)] = dv * _OUTW + jval
            pltpu.async_copy(srcr_v.at[slot], raw_hbm.at[idxr_v.at[slot]],
                             ssem)
            return mc + 1

        def process_block(c, par, mc):
            c_off = blk_off(c)
            wait_blk(par)

            def k_body(k, mc):
                rv = r_v[pl.ds(c_off + k * 16, 16)]
                m0 = rv > 0

                def have(args):
                    m, mc = args

                    def w_cond(st):
                        m, _ = st
                        return plsc.all_reduce_population_count(m)[0] > 0

                    def w_body(st):
                        m, mc = st
                        f = plsc.all_reduce_ffs(m)[0]
                        jval = jnp.sum(
                            jnp.where(iota == f, rv, 0)) - 1
                        mc = extract_one(k * 16 + f, jval, par, mc)
                        return m & (iota != f), mc

                    _, mc = lax.while_loop(w_cond, w_body, (m, mc))
                    return mc

                return lax.cond(
                    plsc.all_reduce_population_count(m0)[0] > 0,
                    have, lambda a: a[1], (m0, mc))

            mc = lax.fori_loop(0, _BW // 16, k_body, mc)

            blk_lo = lo_id + c_off

            def dup_scan(q, mc):
                dids = dupi_v[pl.ds(q * 16, 16)]
                jds = dupd_v[pl.ds(q * 16, 16)]
                valid = (q * 16 + iota) < dupcnt
                m0 = valid & (dids >= blk_lo) & (dids < blk_lo + _BW)

                def have(args):
                    m, mc = args

                    def w_cond(st):
                        m, _ = st
                        return plsc.all_reduce_population_count(m)[0] > 0

                    def w_body(st):
                        m, mc = st
                        f = plsc.all_reduce_ffs(m)[0]
                        sel = iota == f
                        l = jnp.sum(jnp.where(sel, dids, 0)) - blk_lo
                        jval = jnp.sum(jnp.where(sel, jds, 0))
                        mc = extract_one(l, jval, par, mc)
                        return m & ~sel, mc

                    _, mc = lax.while_loop(w_cond, w_body, (m, mc))
                    return mc

                return lax.cond(
                    plsc.all_reduce_population_count(m0)[0] > 0,
                    have, lambda a: a[1], (m0, mc))

            mc = lax.fori_loop(0, (dupcnt + 15) // 16, dup_scan, mc)

            @pl.when(c + 2 < nblk)
            def _():
                start_blk(c + 2, par)

            return mc

        def pair_body(t, mc):
            c0 = t * 2
            mc = process_block(c0, 0, mc)
            return lax.cond(c0 + 1 < nblk,
                            lambda m: process_block(c0 + 1, 1, m),
                            lambda m: m, mc)

        mc = lax.fori_loop(0, (nblk + 1) // 2, pair_body, jnp.int32(0))

        def drain_body(i, _):
            @pl.when(i < jnp.minimum(mc, _RING))
            def _():
                pltpu.make_async_copy(
                    raw_hbm.at[pl.ds(0, _D)], dummy_v, ssem).wait()
            return 0
        lax.fori_loop(0, _RING, drain_body, 0)

    return sc_kernel


_sc_kernel = _make_sc_kernel()


def _tc_norm_kernel(rawT_ref, out_ref):
    x = rawT_ref[...]
    ss = jnp.sum(x * x, axis=0, keepdims=True)
    scale = jnp.where(ss > _MAX_NORM * _MAX_NORM,
                      _MAX_NORM / (jnp.sqrt(ss) + 1e-7),
                      jnp.float32(1.0))
    out_ref[...] = x * scale


_TC_BLK = 2048

_tc_norm = pl.pallas_call(
    _tc_norm_kernel,
    grid=(_B // _TC_BLK,),
    in_specs=[pl.BlockSpec((_D, _TC_BLK), lambda i: (0, i))],
    out_specs=pl.BlockSpec((_D, _TC_BLK), lambda i: (0, i)),
    out_shape=jax.ShapeDtypeStruct((_D, _B), jnp.float32),
)


@jax.jit
def kernel(node_ids, table):
    raw = _sc_kernel(node_ids, table.T)
    rawT = raw.reshape(_D, _OUTW)
    outT = _tc_norm(rawT)
    return outT.T


# speedup vs baseline: 1.0009x; 1.0009x over previous
import functools

import jax
import jax.numpy as jnp
from jax import lax
from jax.experimental import pallas as pl
from jax.experimental.pallas import tpu as pltpu
from jax.experimental.pallas import tpu_sc as plsc

_V = 1000000
_D = 64
_MAX_NORM = 1.0
_B = 16384

_NW = 32
_NCOLS = (_V + 127) // 128
_CPW = (_NCOLS + _NW - 1) // _NW
_IDS_CHUNK = 2048
_OUTW = _B
_RING = 64
_BW = 256


def _iota16():
    return jnp.arange(16, dtype=jnp.int32)


def _make_sc_kernel():
    mesh = plsc.VectorSubcoreMesh(core_axis_name="c", subcore_axis_name="s")

    @functools.partial(
        pl.kernel,
        mesh=mesh,
        compiler_params=pltpu.CompilerParams(
            use_tc_tiling_on_sc=True, needs_layout_passes=False),
        out_type=jax.ShapeDtypeStruct((_D * _OUTW,), jnp.float32),
        scratch_types=[
            pltpu.VMEM((2, _D, _BW), jnp.float32),
            pltpu.VMEM((_CPW * 128,), jnp.int32),
            pltpu.VMEM((_IDS_CHUNK,), jnp.int32),
            pltpu.VMEM((_RING, _D), jnp.float32),
            pltpu.VMEM((_RING, _D), jnp.int32),
            pltpu.VMEM((_B,), jnp.int32),
            pltpu.VMEM((_B,), jnp.int32),
            pltpu.VMEM((_D,), jnp.float32),
            pltpu.SemaphoreType.DMA,
            pltpu.SemaphoreType.DMA,
            pltpu.SemaphoreType.DMA,
        ],
    )
    def sc_kernel(ids_hbm, tabT_hbm, raw_hbm, blk_v, r_v, ids_v,
                  srcr_v, idxr_v, dupd_v, dupi_v, dummy_v,
                  bsem0, bsem1, ssem):
        wid = lax.axis_index("s") * 2 + lax.axis_index("c")
        lo_col = wid * _CPW
        hi_col = jnp.minimum(lo_col + _CPW, _NCOLS)
        ncols = hi_col - lo_col
        lo_id = lo_col * 128
        hi_id = jnp.minimum(hi_col * 128, _V)
        iota = _iota16()
        zero16 = jnp.zeros((16,), jnp.int32)

        def z_body(i, _):
            for u in range(4):
                r_v[pl.ds((i * 4 + u) * 16, 16)] = zero16
            return 0
        lax.fori_loop(0, _CPW * 128 // 64, z_body, 0)

        def scan1_chunk(g, _):
            pltpu.sync_copy(ids_hbm.at[pl.ds(g * _IDS_CHUNK, _IDS_CHUNK)],
                            ids_v)
            def scan1_vec(k, _):
                for u in range(4):
                    o = (k * 4 + u) * 16
                    v = ids_v[pl.ds(o, 16)]
                    jv = g * _IDS_CHUNK + o + iota
                    m = (v >= lo_id) & (v < hi_id)
                    plsc.store_scatter(r_v, [v - lo_id], jv + 1, mask=m)
                return 0
            lax.fori_loop(0, _IDS_CHUNK // 64, scan1_vec, 0)
            return 0
        lax.fori_loop(0, _B // _IDS_CHUNK, scan1_chunk, 0)

        def scan2_chunk(g, dcnt):
            pltpu.sync_copy(ids_hbm.at[pl.ds(g * _IDS_CHUNK, _IDS_CHUNK)],
                            ids_v)
            def scan2_vec(k, dcnt):
                for u in range(4):
                    o = (k * 4 + u) * 16
                    v = ids_v[pl.ds(o, 16)]
                    jv = g * _IDS_CHUNK + o + iota
                    m = (v >= lo_id) & (v < hi_id)
                    rb = plsc.load_gather(r_v,
                                          [jnp.where(m, v - lo_id, 0)])
                    coll = m & (rb != jv + 1)
                    pos = plsc.cumsum(coll.astype(jnp.int32)) - 1 + dcnt
                    plsc.store_scatter(dupd_v, [pos], jv, mask=coll)
                    plsc.store_scatter(dupi_v, [pos], v, mask=coll)
                    n = plsc.all_reduce_population_count(coll)
                    dcnt = dcnt + n[0]
                return dcnt
            return lax.fori_loop(0, _IDS_CHUNK // 64, scan2_vec, dcnt)
        dupcnt = lax.fori_loop(0, _B // _IDS_CHUNK, scan2_chunk,
                               jnp.int32(0))

        _BSEMS = (bsem0, bsem1)
        span = ncols * 128
        nblk = (span + _BW - 1) // _BW

        def blk_off(c):
            return pl.multiple_of(jnp.minimum(c * _BW, span - _BW), 128)

        def start_blk(c, par):
            col = pl.multiple_of(lo_id + blk_off(c), 128)
            return pltpu.async_copy(
                tabT_hbm.at[:, pl.ds(col, _BW)], blk_v.at[par],
                _BSEMS[par])

        def wait_blk(par):
            col0 = pl.multiple_of(0, 128)
            pltpu.make_async_copy(
                tabT_hbm.at[:, pl.ds(col0, _BW)], blk_v.at[par],
                _BSEMS[par]).wait()

        @pl.when(nblk > 0)
        def _():
            start_blk(0, 0)

        @pl.when(nblk > 1)
        def _():
            start_blk(1, 1)

        def extract_one(l, jval, par, mc):
            slot = lax.rem(mc, _RING)

            @pl.when(mc >= _RING)
            def _():
                pltpu.make_async_copy(
                    raw_hbm.at[pl.ds(0, _D)], dummy_v, ssem).wait()

            lvec = jnp.full((16,), l, jnp.int32)
            for q in range(4):
                dv = iota + q * 16
                srcr_v[slot, pl.ds(q * 16, 16)] = plsc.load_gather(
                    blk_v.at[par], [dv, lvec])
                idxr_v[slot, pl.ds(q * 16, 16)] = dv * _OUTW + jval
            pltpu.async_copy(srcr_v.at[slot], raw_hbm.at[idxr_v.at[slot]],
                             ssem)
            return mc + 1

        def process_block(c, par, mc):
            c_off = blk_off(c)
            wait_blk(par)

            def k_body(k, mc):
                rv = r_v[pl.ds(c_off + k * 16, 16)]

                def w_cond(st):
                    m, _ = st
                    return plsc.all_reduce_population_count(m)[0] > 0

                def w_body(st):
                    m, mc = st
                    f = plsc.all_reduce_ffs(m)[0]
                    jval = jnp.sum(jnp.where(iota == f, rv, 0)) - 1
                    mc = extract_one(k * 16 + f, jval, par, mc)
                    return m & (iota != f), mc

                _, mc = lax.while_loop(w_cond, w_body, (rv > 0, mc))
                return mc

            mc = lax.fori_loop(0, _BW // 16, k_body, mc)

            blk_lo = lo_id + c_off

            def dup_scan(q, mc):
                dids = dupi_v[pl.ds(q * 16, 16)]
                jds = dupd_v[pl.ds(q * 16, 16)]
                valid = (q * 16 + iota) < dupcnt
                m0 = valid & (dids >= blk_lo) & (dids < blk_lo + _BW)

                def w_cond(st):
                    m, _ = st
                    return plsc.all_reduce_population_count(m)[0] > 0

                def w_body(st):
                    m, mc = st
                    f = plsc.all_reduce_ffs(m)[0]
                    sel = iota == f
                    l = jnp.sum(jnp.where(sel, dids, 0)) - blk_lo
                    jval = jnp.sum(jnp.where(sel, jds, 0))
                    mc = extract_one(l, jval, par, mc)
                    return m & ~sel, mc

                _, mc = lax.while_loop(w_cond, w_body, (m0, mc))
                return mc

            mc = lax.fori_loop(0, (dupcnt + 15) // 16, dup_scan, mc)

            @pl.when(c + 2 < nblk)
            def _():
                start_blk(c + 2, par)

            return mc

        def pair_body(t, mc):
            c0 = t * 2
            mc = process_block(c0, 0, mc)
            return lax.cond(c0 + 1 < nblk,
                            lambda m: process_block(c0 + 1, 1, m),
                            lambda m: m, mc)

        mc = lax.fori_loop(0, (nblk + 1) // 2, pair_body, jnp.int32(0))

        def drain_body(i, _):
            @pl.when(i < jnp.minimum(mc, _RING))
            def _():
                pltpu.make_async_copy(
                    raw_hbm.at[pl.ds(0, _D)], dummy_v, ssem).wait()
            return 0
        lax.fori_loop(0, _RING, drain_body, 0)

    return sc_kernel


_sc_kernel = _make_sc_kernel()


def _tc_norm_kernel(rawT_ref, out_ref):
    x = rawT_ref[...]
    ss = jnp.sum(x * x, axis=0, keepdims=True)
    scale = jnp.where(ss > _MAX_NORM * _MAX_NORM,
                      _MAX_NORM / (jnp.sqrt(ss) + 1e-7),
                      jnp.float32(1.0))
    out_ref[...] = x * scale


_TC_BLK = 2048

_tc_norm = pl.pallas_call(
    _tc_norm_kernel,
    grid=(_B // _TC_BLK,),
    in_specs=[pl.BlockSpec((_D, _TC_BLK), lambda i: (0, i))],
    out_specs=pl.BlockSpec((_D, _TC_BLK), lambda i: (0, i)),
    out_shape=jax.ShapeDtypeStruct((_D, _B), jnp.float32),
)


@jax.jit
def kernel(node_ids, table):
    raw = _sc_kernel(node_ids, table.T)
    rawT = raw.reshape(_D, _OUTW)
    outT = _tc_norm(rawT)
    return outT.T


# speedup vs baseline: 1.7772x; 1.7756x over previous
import functools

import jax
import jax.numpy as jnp
from jax import lax
from jax.experimental import pallas as pl
from jax.experimental.pallas import tpu as pltpu
from jax.experimental.pallas import tpu_sc as plsc

_MAX_NODE_COUNT = 1000000
_EMBED_DIM = 64
_MAX_NORM = 1.0
_BATCH = 16384

_NC = 2
_NS = 16
_NW = _NC * _NS
_B_PER_W = _BATCH // _NW
_CHUNK = 128
_N_CHUNKS = _B_PER_W // _CHUNK


def _rsqrt_newton(x):
    i = lax.bitcast_convert_type(x, jnp.int32)
    i = jnp.int32(0x5F3759DF) - (i >> 1)
    y = lax.bitcast_convert_type(i, jnp.float32)
    for _ in range(3):
        y = y * (1.5 - 0.5 * x * y * y)
    return y


def _make_kernel():
    mesh = plsc.VectorSubcoreMesh(core_axis_name="c", subcore_axis_name="s")

    @functools.partial(
        pl.kernel,
        mesh=mesh,
        compiler_params=pltpu.CompilerParams(
            needs_layout_passes=False, use_tc_tiling_on_sc=False),
        out_type=jax.ShapeDtypeStruct((_BATCH, _EMBED_DIM), jnp.float32),
        scratch_types=[
            pltpu.VMEM((_B_PER_W,), jnp.int32),
            pltpu.VMEM((_B_PER_W, _EMBED_DIM), jnp.float32),
            pltpu.SemaphoreType.DMA,
        ],
    )
    def emb_kernel(ids_hbm, table_hbm, out_hbm, idx_v, rows_v, sem):
        wid = lax.axis_index("s") * _NC + lax.axis_index("c")
        base = wid * _B_PER_W

        pltpu.sync_copy(ids_hbm.at[pl.ds(base, _B_PER_W)], idx_v)

        copies = []
        for c in range(_N_CHUNKS):
            copies.append(
                pltpu.async_copy(
                    table_hbm.at[idx_v.at[pl.ds(c * _CHUNK, _CHUNK)]],
                    rows_v.at[pl.ds(c * _CHUNK, _CHUNK)],
                    sem,
                )
            )
        for cp in copies:
            cp.wait()

        def row_body(r, _):
            v0 = rows_v[r, pl.ds(0, 16)]
            v1 = rows_v[r, pl.ds(16, 16)]
            v2 = rows_v[r, pl.ds(32, 16)]
            v3 = rows_v[r, pl.ds(48, 16)]
            acc = v0 * v0 + v1 * v1 + v2 * v2 + v3 * v3
            ssq = jnp.sum(acc)
            scale = jnp.where(ssq > _MAX_NORM * _MAX_NORM,
                              _rsqrt_newton(ssq) * _MAX_NORM,
                              jnp.float32(1.0))
            rows_v[r, pl.ds(0, 16)] = v0 * scale
            rows_v[r, pl.ds(16, 16)] = v1 * scale
            rows_v[r, pl.ds(32, 16)] = v2 * scale
            rows_v[r, pl.ds(48, 16)] = v3 * scale
            return 0

        lax.fori_loop(0, _B_PER_W, row_body, 0)

        pltpu.sync_copy(rows_v, out_hbm.at[pl.ds(base, _B_PER_W)])

    return emb_kernel


_emb_kernel = _make_kernel()


@jax.jit
def kernel(node_ids, table):
    return _emb_kernel(node_ids, table)
